# baseline (speedup 1.0000x reference)
import jax
import jax.numpy as jnp
from jax import lax
from jax.experimental import pallas as pl
from jax.experimental.pallas import tpu as pltpu
from jax.experimental.pallas import tpu_sc as plsc

_IGNORE = 255
_N, _C = 65536, 512
_NW = 32
_CHUNK = _N // _NW
_VPW = _CHUNK // 16
_GCH = 128
_NG = _CHUNK // _GCH


def _sc_body(logits_hbm, tgt_hbm, out_hbm, tgt_v, flat_v, gath_v, part_v, sem):
    cid = lax.axis_index("c")
    sid = lax.axis_index("s")
    wid = sid * 2 + cid
    base = wid * _CHUNK

    pltpu.sync_copy(tgt_hbm.at[pl.ds(base, _CHUNK)], tgt_v)

    lanes = lax.iota(jnp.int32, 16)

    def idx_body(j, _):
        t = tgt_v[pl.ds(j * 16, 16)]
        safe = jnp.where(t != _IGNORE, t, 0)
        rows = base + j * 16 + lanes
        flat_v[pl.ds(j * 16, 16)] = rows * _C + safe
        return 0

    lax.fori_loop(0, _VPW, idx_body, 0)

    copies = [
        pltpu.async_copy(
            logits_hbm.at[flat_v.at[pl.ds(k * _GCH, _GCH)]],
            gath_v.at[pl.ds(k * _GCH, _GCH)],
            sem,
        )
        for k in range(_NG)
    ]
    for cp in copies:
        cp.wait()

    zero = jnp.zeros((16,), jnp.float32)

    def red_body(j, carry):
        s, c = carry
        g = gath_v[pl.ds(j * 16, 16)]
        t = tgt_v[pl.ds(j * 16, 16)]
        m = t != _IGNORE
        d = 1.0 - g
        s = s + jnp.where(m, d * d, 0.0)
        c = c + jnp.where(m, 1.0, 0.0)
        return (s, c)

    s, c = lax.fori_loop(0, _VPW, red_body, (zero, zero))
    part_v[0, :] = s
    part_v[1, :] = c
    pltpu.sync_copy(part_v, out_hbm.at[pl.ds(wid * 2, 2)])


_sc_call = pl.kernel(
    _sc_body,
    out_type=jax.ShapeDtypeStruct((2 * _NW, 16), jnp.float32),
    scratch_types=[
        pltpu.VMEM((_CHUNK,), jnp.int32),
        pltpu.VMEM((_CHUNK,), jnp.int32),
        pltpu.VMEM((_CHUNK,), jnp.float32),
        pltpu.VMEM((2, 16), jnp.float32),
        pltpu.SemaphoreType.DMA,
    ],
    mesh=plsc.VectorSubcoreMesh(core_axis_name="c", subcore_axis_name="s"),
)


@jax.jit
def kernel(contrast_logits, contrast_target):
    partials = _sc_call(contrast_logits.reshape(-1), contrast_target)
    total = jnp.sum(partials[0::2])
    count = jnp.sum(partials[1::2])
    return total / jnp.maximum(count, 1.0)


# speedup vs baseline: 4.3128x; 4.3128x over previous
import jax
import jax.numpy as jnp
from jax import lax
from jax.experimental import pallas as pl
from jax.experimental.pallas import tpu as pltpu
from jax.experimental.pallas import tpu_sc as plsc

_IGNORE = 255
_N, _C = 65536, 512
_NW = 32
_CHUNK = _N // _NW
_VPW = _CHUNK // 16
_GCH = 128
_NG = _CHUNK // _GCH


def _sc_body(logits_hbm, tgt_hbm, out_hbm, tgt_v, flat_v, gath_v, part_v, sem):
    cid = lax.axis_index("c")
    sid = lax.axis_index("s")
    wid = sid * 2 + cid
    base = wid * _CHUNK

    pltpu.sync_copy(tgt_hbm.at[pl.ds(base, _CHUNK)], tgt_v)

    lanes = lax.iota(jnp.int32, 16)

    def idx_body(j, _):
        t = tgt_v[pl.ds(j * 16, 16)]
        safe = jnp.where(t != _IGNORE, t, 0)
        rows = base + j * 16 + lanes
        off = ((rows >> 3) * (_C // 128) + (safe >> 7)) * 1024 \
            + (rows & 7) * 128 + (safe & 127)
        flat_v[pl.ds(j * 16, 16)] = off
        return 0

    lax.fori_loop(0, _VPW, idx_body, 0)

    copies = [
        pltpu.async_copy(
            logits_hbm.at[flat_v.at[pl.ds(k * _GCH, _GCH)]],
            gath_v.at[pl.ds(k * _GCH, _GCH)],
            sem,
        )
        for k in range(_NG)
    ]
    for cp in copies:
        cp.wait()

    zero = jnp.zeros((16,), jnp.float32)

    def red_body(j, carry):
        s, c = carry
        g = gath_v[pl.ds(j * 16, 16)]
        t = tgt_v[pl.ds(j * 16, 16)]
        m = t != _IGNORE
        d = 1.0 - g
        s = s + jnp.where(m, d * d, 0.0)
        c = c + jnp.where(m, 1.0, 0.0)
        return (s, c)

    s, c = lax.fori_loop(0, _VPW, red_body, (zero, zero))
    part_v[0, :] = s
    part_v[1, :] = c
    pltpu.sync_copy(part_v, out_hbm.at[pl.ds(wid * 2, 2)])


_sc_call = pl.kernel(
    _sc_body,
    out_type=jax.ShapeDtypeStruct((2 * _NW, 16), jnp.float32),
    scratch_types=[
        pltpu.VMEM((_CHUNK,), jnp.int32),
        pltpu.VMEM((_CHUNK,), jnp.int32),
        pltpu.VMEM((_CHUNK,), jnp.float32),
        pltpu.VMEM((2, 16), jnp.float32),
        pltpu.SemaphoreType.DMA,
    ],
    mesh=plsc.VectorSubcoreMesh(core_axis_name="c", subcore_axis_name="s"),
)


@jax.jit
def kernel(contrast_logits, contrast_target):
    tiles = contrast_logits.reshape(_N // 8, 8, _C // 128, 128)
    flat = tiles.transpose(0, 2, 1, 3).reshape(-1)
    partials = _sc_call(flat, contrast_target)
    total = jnp.sum(partials[0::2])
    count = jnp.sum(partials[1::2])
    return total / jnp.maximum(count, 1.0)
